# baseline (speedup 1.0000x reference)
import functools

import jax
import jax.numpy as jnp
from jax import lax
from jax.experimental import pallas as pl
from jax.experimental.pallas import tpu as pltpu
from jax.experimental.pallas import tpu_sc as plsc

_CHUNK = 128


def _make_gather(num_rows, d, b, nw):
  assert b % (nw * _CHUNK) == 0
  chunks_per_w = b // (nw * _CHUNK)
  mesh = plsc.VectorSubcoreMesh(core_axis_name="c", subcore_axis_name="s")
  nc = 2

  @functools.partial(
      pl.kernel,
      mesh=mesh,
      out_type=jax.ShapeDtypeStruct((b, d), jnp.float32),
      scratch_types=[
          pltpu.VMEM((chunks_per_w, _CHUNK), jnp.int32),
          pltpu.VMEM((_CHUNK, d), jnp.float32),
          pltpu.VMEM((_CHUNK, d), jnp.float32),
          pltpu.SemaphoreType.DMA,
          pltpu.SemaphoreType.DMA,
      ],
      compiler_params=pltpu.CompilerParams(use_tc_tiling_on_sc=False),
  )
  def gather_kernel(table_hbm, idx_hbm, out_hbm, idx_v, rows0, rows1, sem0,
                    sem1):
    wid = lax.axis_index("s") * nc + lax.axis_index("c")
    base = wid * (chunks_per_w * _CHUNK)
    pltpu.sync_copy(idx_hbm.at[wid], idx_v)

    bufs = (rows0, rows1)
    sems = (sem0, sem1)
    nbuf = 2

    def issue(j, p):
      pltpu.async_copy(table_hbm.at[idx_v.at[j]], bufs[p], sems[p])

    def drain(j, p):
      pltpu.make_async_copy(table_hbm.at[idx_v.at[j]], bufs[p], sems[p]).wait()
      off = pl.multiple_of(base + j * _CHUNK, _CHUNK)
      pltpu.sync_copy(bufs[p], out_hbm.at[pl.ds(off, _CHUNK)])

    for p in range(nbuf):
      issue(p, p)

    def body(g, carry):
      del carry
      j = g * nbuf
      for p in range(nbuf):
        drain(j + p, p)

        @pl.when(j + p + nbuf < chunks_per_w)
        def _():
          issue(j + p + nbuf, p)

      return 0

    assert chunks_per_w % nbuf == 0
    lax.fori_loop(0, chunks_per_w // nbuf, body, 0)

  return gather_kernel


def kernel(input_tensor, dim, indices):
  data = input_tensor
  dim_size = data.shape[0]
  d = data.shape[1]
  original_shape = indices.shape
  flat_idx = indices.reshape(-1).astype(jnp.int32) + jnp.asarray(
      dim, dtype=jnp.int32)
  flat_idx = jnp.where(flat_idx < 0, flat_idx + dim_size, flat_idx)
  b = flat_idx.shape[0]

  info = plsc.get_sparse_core_info()
  nw = info.num_cores * info.num_subcores
  idx3 = flat_idx.reshape(nw, b // (nw * _CHUNK), _CHUNK)
  out = _make_gather(dim_size, d, b, nw)(data.astype(jnp.float32), idx3)
  return out.reshape(*original_shape, d).astype(data.dtype)


# speedup vs baseline: 1.0088x; 1.0088x over previous
import functools

import jax
import jax.numpy as jnp
from jax import lax
from jax.experimental import pallas as pl
from jax.experimental.pallas import tpu as pltpu
from jax.experimental.pallas import tpu_sc as plsc

_CHUNK = 128
_NBUF = 5


def _make_gather(num_rows, d, b, nw):
  assert b % (nw * _CHUNK) == 0
  chunks_per_w = b // (nw * _CHUNK)
  mesh = plsc.VectorSubcoreMesh(core_axis_name="c", subcore_axis_name="s")
  nc = 2

  @functools.partial(
      pl.kernel,
      mesh=mesh,
      out_type=jax.ShapeDtypeStruct((b, d), jnp.float32),
      scratch_types=[
          pltpu.VMEM((chunks_per_w, _CHUNK), jnp.int32),
      ] + [pltpu.VMEM((_CHUNK, d), jnp.float32) for _ in range(_NBUF)] + [
          pltpu.SemaphoreType.DMA for _ in range(_NBUF)
      ],
      compiler_params=pltpu.CompilerParams(use_tc_tiling_on_sc=False),
  )
  def gather_kernel(table_hbm, idx_hbm, out_hbm, idx_v, *bufs_sems):
    wid = lax.axis_index("s") * nc + lax.axis_index("c")
    base = wid * (chunks_per_w * _CHUNK)
    pltpu.sync_copy(idx_hbm.at[wid], idx_v)

    bufs = bufs_sems[:_NBUF]
    sems = bufs_sems[_NBUF:]
    depth = _NBUF - 1

    def issue(j, p):
      pltpu.async_copy(table_hbm.at[idx_v.at[j]], bufs[p], sems[p])

    def drain(j, p):
      pltpu.make_async_copy(table_hbm.at[idx_v.at[j]], bufs[p], sems[p]).wait()
      off = pl.multiple_of(base + j * _CHUNK, _CHUNK)
      pltpu.sync_copy(bufs[p], out_hbm.at[pl.ds(off, _CHUNK)])

    for p in range(depth):
      issue(p, p)

    def body(g, carry):
      del carry
      j = g * _NBUF
      for p in range(_NBUF):
        drain(j + p, p)

        @pl.when(j + p + depth < chunks_per_w)
        def _():
          issue(j + p + depth, (p + depth) % _NBUF)

      return 0

    assert chunks_per_w % _NBUF == 0
    lax.fori_loop(0, chunks_per_w // _NBUF, body, 0)

  return gather_kernel


def kernel(input_tensor, dim, indices):
  data = input_tensor
  dim_size = data.shape[0]
  d = data.shape[1]
  original_shape = indices.shape
  flat_idx = indices.reshape(-1).astype(jnp.int32) + jnp.asarray(
      dim, dtype=jnp.int32)
  flat_idx = jnp.where(flat_idx < 0, flat_idx + dim_size, flat_idx)
  b = flat_idx.shape[0]

  info = plsc.get_sparse_core_info()
  nw = info.num_cores * info.num_subcores
  idx3 = flat_idx.reshape(nw, b // (nw * _CHUNK), _CHUNK)
  out = _make_gather(dim_size, d, b, nw)(data.astype(jnp.float32), idx3)
  return out.reshape(*original_shape, d).astype(data.dtype)
